# baseline (speedup 1.0000x reference)
import functools

import jax
import jax.numpy as jnp
from jax import lax
from jax.experimental import pallas as pl
from jax.experimental.pallas import tpu as pltpu
from jax.experimental.pallas import tpu_sc as plsc

N = 50000
E = 600000
D_IN = 128
H = 64
HH = 32

NC = 2
NS = 16
CHUNK = 128
ROWS = 296
PER_TILE = ROWS * CHUNK
E_PAD = NS * PER_TILE
ACC_ROWS = 50048
ZROWS = ACC_ROWS // NS
NSLOT = 4
IC = 37
NCHUNK = ROWS // IC

_MESH = plsc.VectorSubcoreMesh(
    core_axis_name="c", subcore_axis_name="s", num_cores=NC, num_subcores=NS)
_SC_PARAMS = pltpu.CompilerParams(use_tc_tiling_on_sc=False)

BLK = 2000
GRID = N // BLK


def _row_spec(w):
    return pl.BlockSpec((BLK, w), lambda i: (i, 0))


def _full_spec(shape):
    return pl.BlockSpec(shape, lambda i: tuple(0 for _ in shape))


def _half_spec():
    return pl.BlockSpec((2, BLK, HH), lambda i: (0, i, 0))


def _cat_halves(s):
    return jnp.concatenate([s[0], s[1]], axis=1)


def _prep_body(xd, xg, wl_dg, wr_dg, wl_gd, wr_gd, b_dg, b_gd,
               pg_lo, pg_hi, pd_lo, pd_hi, rg, rd):
    pg = jnp.dot(xd[...], wl_dg[...], preferred_element_type=jnp.float32)
    pg_lo[...] = pg[:, :HH]
    pg_hi[...] = pg[:, HH:]
    pd = jnp.dot(xg[...], wl_gd[...], preferred_element_type=jnp.float32)
    pd_lo[...] = pd[:, :HH]
    pd_hi[...] = pd[:, HH:]
    rg[...] = jnp.dot(xg[...], wr_dg[...], preferred_element_type=jnp.float32) + b_dg[...]
    rd[...] = jnp.dot(xd[...], wr_gd[...], preferred_element_type=jnp.float32) + b_gd[...]


def _prep(xd, xg, wl_dg, wr_dg, wl_gd, wr_gd, b_dg, b_gd):
    f32 = jnp.float32
    return pl.pallas_call(
        _prep_body,
        grid=(GRID,),
        in_specs=[_row_spec(D_IN), _row_spec(D_IN),
                  _full_spec((D_IN, H)), _full_spec((D_IN, H)),
                  _full_spec((D_IN, H)), _full_spec((D_IN, H)),
                  _full_spec((1, H)), _full_spec((1, H))],
        out_specs=[_row_spec(HH)] * 4 + [_row_spec(H), _row_spec(H)],
        out_shape=[jax.ShapeDtypeStruct((N, HH), f32)] * 4
                  + [jax.ShapeDtypeStruct((N, H), f32)] * 2,
    )(xd, xg, wl_dg, wr_dg, wl_gd, wr_gd, b_dg, b_gd)


def _mid_body(s1g, s1d, cg, cd, r1g, r1d,
              wl2_dg, wr2_dg, wl2_gd, wr2_gd, b2_dg, b2_gd,
              pg_lo, pg_hi, pd_lo, pd_hi, r2g, r2d):
    g1 = _cat_halves(s1g) / jnp.maximum(cg[...], 1.0) + r1g[...]
    d1 = _cat_halves(s1d) / jnp.maximum(cd[...], 1.0) + r1d[...]
    p2g = jnp.dot(d1, wl2_dg[...], preferred_element_type=jnp.float32)
    pg_lo[...] = p2g[:, :HH]
    pg_hi[...] = p2g[:, HH:]
    p2d = jnp.dot(g1, wl2_gd[...], preferred_element_type=jnp.float32)
    pd_lo[...] = p2d[:, :HH]
    pd_hi[...] = p2d[:, HH:]
    r2g[...] = jnp.dot(g1, wr2_dg[...], preferred_element_type=jnp.float32) + b2_dg[...]
    r2d[...] = jnp.dot(d1, wr2_gd[...], preferred_element_type=jnp.float32) + b2_gd[...]


def _mid(s1g, s1d, cg, cd, r1g, r1d, wl2_dg, wr2_dg, wl2_gd, wr2_gd, b2_dg, b2_gd):
    f32 = jnp.float32
    return pl.pallas_call(
        _mid_body,
        grid=(GRID,),
        in_specs=[_half_spec(), _half_spec(),
                  _row_spec(1), _row_spec(1), _row_spec(H), _row_spec(H),
                  _full_spec((H, H)), _full_spec((H, H)),
                  _full_spec((H, H)), _full_spec((H, H)),
                  _full_spec((1, H)), _full_spec((1, H))],
        out_specs=[_row_spec(HH)] * 4 + [_row_spec(H), _row_spec(H)],
        out_shape=[jax.ShapeDtypeStruct((N, HH), f32)] * 4
                  + [jax.ShapeDtypeStruct((N, H), f32)] * 2,
    )(s1g, s1d, cg, cd, r1g, r1d, wl2_dg, wr2_dg, wl2_gd, wr2_gd, b2_dg, b2_gd)


def _final_body(s2g, s2d, cg, cd, r2g, r2d, d2, g2):
    g2[...] = _cat_halves(s2g) / jnp.maximum(cg[...], 1.0) + r2g[...]
    d2[...] = _cat_halves(s2d) / jnp.maximum(cd[...], 1.0) + r2d[...]


def _final(s2g, s2d, cg, cd, r2g, r2d):
    f32 = jnp.float32
    return pl.pallas_call(
        _final_body,
        grid=(GRID,),
        in_specs=[_half_spec(), _half_spec(),
                  _row_spec(1), _row_spec(1), _row_spec(H), _row_spec(H)],
        out_specs=[_row_spec(H), _row_spec(H)],
        out_shape=[jax.ShapeDtypeStruct((N, H), f32)] * 2,
    )(s2g, s2d, cg, cd, r2g, r2d)


def _seg_phase(cid, sid, p_lo, p_hi, src_idx, dst_idx, zeros, out,
               schunk, dchunk, bufs, acc, gsem, ssem, dsem, csem):
    def _stage_src(c):
        pltpu.async_copy(src_idx.at[sid, pl.ds(c * IC, IC)], schunk, csem)

    def _wait_src():
        pltpu.make_async_copy(src_idx.at[sid, pl.ds(0, IC)], schunk,
                              csem).wait()

    def _stage_dst(c, par):
        pltpu.async_copy(dst_idx.at[sid, pl.ds(c * IC, IC)], dchunk.at[par],
                         dsem.at[par])

    def _wait_dst(par):
        pltpu.make_async_copy(dst_idx.at[sid, pl.ds(0, IC)], dchunk.at[par],
                              dsem.at[par]).wait()

    def _gather(r, b):
        @pl.when(cid == 0)
        def _():
            pltpu.async_copy(p_lo.at[schunk.at[r]], bufs.at[b], gsem.at[b])

        @pl.when(cid == 1)
        def _():
            pltpu.async_copy(p_hi.at[schunk.at[r]], bufs.at[b], gsem.at[b])

    def _wait_gather(b):
        pltpu.make_async_copy(p_lo.at[pl.ds(0, CHUNK)], bufs.at[b],
                              gsem.at[b]).wait()

    def _wait_scatter(pb, ppar, pr):
        pltpu.make_async_copy(bufs.at[pb], acc.at[dchunk.at[ppar, pr]],
                              ssem.at[pb]).wait()

    def _run_chunk(c, par, has_next):
        _wait_src()
        _wait_dst(par)

        @pl.when(has_next)
        def _():
            _stage_dst(c + 1, 1 - par)

        for b in range(NSLOT - 1):
            _gather(b, b)
        for r in range(IC):
            b = r % NSLOT
            _wait_gather(b)
            if r == IC - 1:
                @pl.when(has_next)
                def _():
                    _stage_src(c + 1)
            pltpu.async_copy(bufs.at[b], acc.at[dchunk.at[par, r]],
                             ssem.at[b], add=True)
            g = r + NSLOT - 1
            if g < IC:
                gb = g % NSLOT
                if g >= NSLOT:
                    _wait_scatter(gb, par, g - NSLOT)
                _gather(g, gb)
        for r in range(IC - NSLOT, IC):
            _wait_scatter(r % NSLOT, par, r)

    pltpu.sync_copy(zeros, acc.at[pl.ds(sid * ZROWS, ZROWS)])
    _stage_src(jnp.int32(0))
    _stage_dst(jnp.int32(0), 0)
    plsc.subcore_barrier()

    def _body(jo, carry):
        c0 = jo * 2
        _run_chunk(c0, 0, c0 + 1 < NCHUNK)
        _run_chunk(c0 + 1, 1, c0 + 2 < NCHUNK)
        return carry

    lax.fori_loop(0, NCHUNK // 2, _body, 0, unroll=False)
    plsc.subcore_barrier()
    pltpu.sync_copy(acc.at[pl.ds(sid * ZROWS, ZROWS)],
                    out.at[cid, pl.ds(sid * ZROWS, ZROWS)])


def _cnt_phase(cid, sid, ddg, dgd, ones_v, zeros, out,
               dchunk, acc, dsem, ssem0):
    def _stage(c, par):
        @pl.when(cid == 0)
        def _():
            pltpu.async_copy(ddg.at[sid, pl.ds(c * IC, IC)], dchunk.at[par],
                             dsem.at[par])

        @pl.when(cid == 1)
        def _():
            pltpu.async_copy(dgd.at[sid, pl.ds(c * IC, IC)], dchunk.at[par],
                             dsem.at[par])

    def _wait_stage(par):
        pltpu.make_async_copy(ddg.at[sid, pl.ds(0, IC)], dchunk.at[par],
                              dsem.at[par]).wait()

    def _drain_scatter(ppar, pr):
        pltpu.make_async_copy(ones_v, acc.at[dchunk.at[ppar, pr]],
                              ssem0).wait()

    pltpu.sync_copy(zeros, acc.at[pl.ds(sid * ZROWS, ZROWS)])
    _stage(jnp.int32(0), 0)
    plsc.subcore_barrier()

    def _run_chunk(c, par, has_next):
        _wait_stage(par)

        @pl.when(has_next)
        def _():
            _stage(c + 1, 1 - par)

        for r in range(IC):
            if r > 0:
                _drain_scatter(par, r - 1)
            pltpu.async_copy(ones_v, acc.at[dchunk.at[par, r]], ssem0,
                             add=True)
        _drain_scatter(par, IC - 1)

    def _body(jo, carry):
        c0 = jo * 2
        _run_chunk(c0, 0, c0 + 1 < NCHUNK)
        _run_chunk(c0 + 1, 1, c0 + 2 < NCHUNK)
        return carry

    lax.fori_loop(0, NCHUNK // 2, _body, 0, unroll=False)
    plsc.subcore_barrier()
    pltpu.sync_copy(acc.at[pl.ds(sid * ZROWS, ZROWS)],
                    out.at[cid, pl.ds(sid * ZROWS, ZROWS)])


_SC_SCRATCH = [
    pltpu.VMEM((IC, CHUNK), jnp.int32),
    pltpu.VMEM((2, IC, CHUNK), jnp.int32),
    pltpu.VMEM((NSLOT, CHUNK, HH), jnp.float32),
    pltpu.VMEM_SHARED((ACC_ROWS, HH), jnp.float32),
    pltpu.SemaphoreType.DMA((NSLOT,)),
    pltpu.SemaphoreType.DMA((NSLOT,)),
    pltpu.SemaphoreType.DMA((2,)),
    pltpu.SemaphoreType.DMA,
]


@functools.partial(
    pl.kernel,
    out_type=[jax.ShapeDtypeStruct((NC, ACC_ROWS, HH), jnp.float32)] * 3,
    mesh=_MESH,
    compiler_params=_SC_PARAMS,
    scratch_types=_SC_SCRATCH,
)
def _layer1(pg_lo, pg_hi, pd_lo, pd_hi, sdg, ddg, sgd, dgd, ones, zeros,
            cnt, s1g, s1d,
            schunk, dchunk, bufs, acc, gsem, ssem, dsem, csem):
    cid = lax.axis_index("c")
    sid = lax.axis_index("s")
    ones_v = bufs.at[0]
    pltpu.sync_copy(ones, ones_v)
    _cnt_phase(cid, sid, ddg, dgd, ones_v, zeros, cnt,
               dchunk, acc, dsem, ssem.at[0])
    _seg_phase(cid, sid, pg_lo, pg_hi, sdg, ddg, zeros, s1g,
               schunk, dchunk, bufs, acc, gsem, ssem, dsem, csem)
    _seg_phase(cid, sid, pd_lo, pd_hi, sgd, dgd, zeros, s1d,
               schunk, dchunk, bufs, acc, gsem, ssem, dsem, csem)


@functools.partial(
    pl.kernel,
    out_type=[jax.ShapeDtypeStruct((NC, ACC_ROWS, HH), jnp.float32)] * 2,
    mesh=_MESH,
    compiler_params=_SC_PARAMS,
    scratch_types=_SC_SCRATCH,
)
def _layer2(pg_lo, pg_hi, pd_lo, pd_hi, sdg, ddg, sgd, dgd, ones, zeros,
            s2g, s2d,
            schunk, dchunk, bufs, acc, gsem, ssem, dsem, csem):
    cid = lax.axis_index("c")
    sid = lax.axis_index("s")
    del ones
    _seg_phase(cid, sid, pg_lo, pg_hi, sdg, ddg, zeros, s2g,
               schunk, dchunk, bufs, acc, gsem, ssem, dsem, csem)
    _seg_phase(cid, sid, pd_lo, pd_hi, sgd, dgd, zeros, s2d,
               schunk, dchunk, bufs, acc, gsem, ssem, dsem, csem)


def _pack_idx(v, fill):
    v = v.astype(jnp.int32)[:min(E, E_PAD)]
    pad = jnp.full((max(E_PAD - E, 0),), fill, jnp.int32)
    return jnp.concatenate([v, pad]).reshape(NS, ROWS, CHUNK)


def kernel(x_disease, x_gene,
           W_l1_dg, b_l1_dg, W_r1_dg,
           W_l1_gd, b_l1_gd, W_r1_gd,
           W_l2_dg, b_l2_dg, W_r2_dg,
           W_l2_gd, b_l2_gd, W_r2_gd,
           edge_index_dg, edge_index_gd):
    f32 = jnp.float32
    sdg = _pack_idx(edge_index_dg[0], 0)
    ddg = _pack_idx(edge_index_dg[1], N)
    sgd = _pack_idx(edge_index_gd[0], 0)
    dgd = _pack_idx(edge_index_gd[1], N)
    zeros = jnp.zeros((ZROWS, HH), f32)
    ones = jnp.ones((CHUNK, HH), f32)

    p1 = _prep(x_disease, x_gene, W_l1_dg, W_r1_dg, W_l1_gd, W_r1_gd,
               b_l1_dg.reshape(1, H), b_l1_gd.reshape(1, H))
    pg_lo, pg_hi, pd_lo, pd_hi, r1g, r1d = p1

    cnt, s1g, s1d = _layer1(pg_lo, pg_hi, pd_lo, pd_hi,
                            sdg, ddg, sgd, dgd, ones, zeros)
    cg = cnt[0, :, 0:1]
    cd = cnt[1, :, 0:1]

    p2 = _mid(s1g, s1d, cg, cd, r1g, r1d,
              W_l2_dg, W_r2_dg, W_l2_gd, W_r2_gd,
              b_l2_dg.reshape(1, H), b_l2_gd.reshape(1, H))
    qg_lo, qg_hi, qd_lo, qd_hi, r2g, r2d = p2

    s2g, s2d = _layer2(qg_lo, qg_hi, qd_lo, qd_hi,
                       sdg, ddg, sgd, dgd, ones, zeros)

    d2, g2 = _final(s2g, s2d, cg, cd, r2g, r2d)
    return (d2, g2)


# speedup vs baseline: 1.0656x; 1.0656x over previous
import functools

import jax
import jax.numpy as jnp
from jax import lax
from jax.experimental import pallas as pl
from jax.experimental.pallas import tpu as pltpu
from jax.experimental.pallas import tpu_sc as plsc

N = 50000
E = 600000
D_IN = 128
H = 64
HH = 32

NC = 2
NS = 16
CHUNK = 128
ROWS = 296
PER_TILE = ROWS * CHUNK
E_PAD = NS * PER_TILE
ACC_ROWS = 50048
ZROWS = ACC_ROWS // NS
NSLOT = 4
IC = 37
NCHUNK = ROWS // IC

_MESH = plsc.VectorSubcoreMesh(
    core_axis_name="c", subcore_axis_name="s", num_cores=NC, num_subcores=NS)
_SC_PARAMS = pltpu.CompilerParams(use_tc_tiling_on_sc=False)

BLK = 2000
GRID = N // BLK


def _row_spec(w):
    return pl.BlockSpec((BLK, w), lambda i: (i, 0))


def _full_spec(shape):
    return pl.BlockSpec(shape, lambda i: tuple(0 for _ in shape))


def _half_spec():
    return pl.BlockSpec((2, BLK, HH), lambda i: (0, i, 0))


def _cat_halves(s):
    return jnp.concatenate([s[0], s[1]], axis=1)


def _prep_body(xd, xg, wl_dg, wr_dg, wl_gd, wr_gd, b_dg, b_gd,
               pg_lo, pg_hi, pd_lo, pd_hi, rg, rd):
    pg = jnp.dot(xd[...], wl_dg[...], preferred_element_type=jnp.float32)
    pg_lo[...] = pg[:, :HH]
    pg_hi[...] = pg[:, HH:]
    pd = jnp.dot(xg[...], wl_gd[...], preferred_element_type=jnp.float32)
    pd_lo[...] = pd[:, :HH]
    pd_hi[...] = pd[:, HH:]
    rg[...] = jnp.dot(xg[...], wr_dg[...], preferred_element_type=jnp.float32) + b_dg[...]
    rd[...] = jnp.dot(xd[...], wr_gd[...], preferred_element_type=jnp.float32) + b_gd[...]


def _prep(xd, xg, wl_dg, wr_dg, wl_gd, wr_gd, b_dg, b_gd):
    f32 = jnp.float32
    return pl.pallas_call(
        _prep_body,
        grid=(GRID,),
        in_specs=[_row_spec(D_IN), _row_spec(D_IN),
                  _full_spec((D_IN, H)), _full_spec((D_IN, H)),
                  _full_spec((D_IN, H)), _full_spec((D_IN, H)),
                  _full_spec((1, H)), _full_spec((1, H))],
        out_specs=[_row_spec(HH)] * 4 + [_row_spec(H), _row_spec(H)],
        out_shape=[jax.ShapeDtypeStruct((N, HH), f32)] * 4
                  + [jax.ShapeDtypeStruct((N, H), f32)] * 2,
    )(xd, xg, wl_dg, wr_dg, wl_gd, wr_gd, b_dg, b_gd)


def _mid_body(s1g, s1d, cg, cd, r1g, r1d,
              wl2_dg, wr2_dg, wl2_gd, wr2_gd, b2_dg, b2_gd,
              pg_lo, pg_hi, pd_lo, pd_hi, r2g, r2d):
    g1 = _cat_halves(s1g) / jnp.maximum(cg[...], 1.0) + r1g[...]
    d1 = _cat_halves(s1d) / jnp.maximum(cd[...], 1.0) + r1d[...]
    p2g = jnp.dot(d1, wl2_dg[...], preferred_element_type=jnp.float32)
    pg_lo[...] = p2g[:, :HH]
    pg_hi[...] = p2g[:, HH:]
    p2d = jnp.dot(g1, wl2_gd[...], preferred_element_type=jnp.float32)
    pd_lo[...] = p2d[:, :HH]
    pd_hi[...] = p2d[:, HH:]
    r2g[...] = jnp.dot(g1, wr2_dg[...], preferred_element_type=jnp.float32) + b2_dg[...]
    r2d[...] = jnp.dot(d1, wr2_gd[...], preferred_element_type=jnp.float32) + b2_gd[...]


def _mid(s1g, s1d, cg, cd, r1g, r1d, wl2_dg, wr2_dg, wl2_gd, wr2_gd, b2_dg, b2_gd):
    f32 = jnp.float32
    return pl.pallas_call(
        _mid_body,
        grid=(GRID,),
        in_specs=[_half_spec(), _half_spec(),
                  _row_spec(1), _row_spec(1), _row_spec(H), _row_spec(H),
                  _full_spec((H, H)), _full_spec((H, H)),
                  _full_spec((H, H)), _full_spec((H, H)),
                  _full_spec((1, H)), _full_spec((1, H))],
        out_specs=[_row_spec(HH)] * 4 + [_row_spec(H), _row_spec(H)],
        out_shape=[jax.ShapeDtypeStruct((N, HH), f32)] * 4
                  + [jax.ShapeDtypeStruct((N, H), f32)] * 2,
    )(s1g, s1d, cg, cd, r1g, r1d, wl2_dg, wr2_dg, wl2_gd, wr2_gd, b2_dg, b2_gd)


def _final_body(s2g, s2d, cg, cd, r2g, r2d, d2, g2):
    g2[...] = _cat_halves(s2g) / jnp.maximum(cg[...], 1.0) + r2g[...]
    d2[...] = _cat_halves(s2d) / jnp.maximum(cd[...], 1.0) + r2d[...]


def _final(s2g, s2d, cg, cd, r2g, r2d):
    f32 = jnp.float32
    return pl.pallas_call(
        _final_body,
        grid=(GRID,),
        in_specs=[_half_spec(), _half_spec(),
                  _row_spec(1), _row_spec(1), _row_spec(H), _row_spec(H)],
        out_specs=[_row_spec(H), _row_spec(H)],
        out_shape=[jax.ShapeDtypeStruct((N, H), f32)] * 2,
    )(s2g, s2d, cg, cd, r2g, r2d)


def _seg_phase(cid, sid, p_lo, p_hi, src_idx, dst_idx, zeros, out,
               schunk, dchunk, bufs, acc, gsem, ssem, dsem, csem):
    def _stage_src(c):
        pltpu.async_copy(src_idx.at[sid, pl.ds(c * IC, IC)], schunk, csem)

    def _wait_src():
        pltpu.make_async_copy(src_idx.at[sid, pl.ds(0, IC)], schunk,
                              csem).wait()

    def _stage_dst(c, par):
        pltpu.async_copy(dst_idx.at[sid, pl.ds(c * IC, IC)], dchunk.at[par],
                         dsem.at[par])

    def _wait_dst(par):
        pltpu.make_async_copy(dst_idx.at[sid, pl.ds(0, IC)], dchunk.at[par],
                              dsem.at[par]).wait()

    def _gather(r, b):
        @pl.when(cid == 0)
        def _():
            pltpu.async_copy(p_lo.at[schunk.at[r]], bufs.at[b], gsem.at[b])

        @pl.when(cid == 1)
        def _():
            pltpu.async_copy(p_hi.at[schunk.at[r]], bufs.at[b], gsem.at[b])

    def _wait_gather(b):
        pltpu.make_async_copy(p_lo.at[pl.ds(0, CHUNK)], bufs.at[b],
                              gsem.at[b]).wait()

    def _wait_scatter(pb, ppar, pr):
        pltpu.make_async_copy(bufs.at[pb], acc.at[dchunk.at[ppar, pr]],
                              ssem.at[pb]).wait()

    def _run_chunk(c, par, has_next):
        _wait_src()
        _wait_dst(par)

        @pl.when(has_next)
        def _():
            _stage_dst(c + 1, 1 - par)

        for b in range(NSLOT - 1):
            _gather(b, b)
        for r in range(IC):
            b = r % NSLOT
            _wait_gather(b)
            if r == IC - 1:
                @pl.when(has_next)
                def _():
                    _stage_src(c + 1)
            pltpu.async_copy(bufs.at[b], acc.at[dchunk.at[par, r]],
                             ssem.at[b], add=True)
            g = r + NSLOT - 1
            if g < IC:
                gb = g % NSLOT
                if g >= NSLOT:
                    _wait_scatter(gb, par, g - NSLOT)
                _gather(g, gb)
        for r in range(IC - NSLOT, IC):
            _wait_scatter(r % NSLOT, par, r)

    pltpu.sync_copy(zeros, acc.at[pl.ds(sid * ZROWS, ZROWS)])
    _stage_src(jnp.int32(0))
    _stage_dst(jnp.int32(0), 0)
    plsc.subcore_barrier()

    def _body(jo, carry):
        c0 = jo * 2
        _run_chunk(c0, 0, c0 + 1 < NCHUNK)
        _run_chunk(c0 + 1, 1, c0 + 2 < NCHUNK)
        return carry

    lax.fori_loop(0, NCHUNK // 2, _body, 0, unroll=False)
    plsc.subcore_barrier()
    pltpu.sync_copy(acc.at[pl.ds(sid * ZROWS, ZROWS)],
                    out.at[cid, pl.ds(sid * ZROWS, ZROWS)])


def _cnt_phase(cid, sid, ddg, dgd, ones_v, zeros, out,
               dchunk, acc, dsem, ssem0):
    def _stage(c, par):
        @pl.when(cid == 0)
        def _():
            pltpu.async_copy(ddg.at[sid, pl.ds(c * IC, IC)], dchunk.at[par],
                             dsem.at[par])

        @pl.when(cid == 1)
        def _():
            pltpu.async_copy(dgd.at[sid, pl.ds(c * IC, IC)], dchunk.at[par],
                             dsem.at[par])

    def _wait_stage(par):
        pltpu.make_async_copy(ddg.at[sid, pl.ds(0, IC)], dchunk.at[par],
                              dsem.at[par]).wait()

    def _drain_scatter(ppar, pr):
        pltpu.make_async_copy(ones_v, acc.at[dchunk.at[ppar, pr]],
                              ssem0).wait()

    pltpu.sync_copy(zeros, acc.at[pl.ds(sid * ZROWS, ZROWS)])
    _stage(jnp.int32(0), 0)
    plsc.subcore_barrier()

    def _run_chunk(c, par, has_next):
        _wait_stage(par)

        @pl.when(has_next)
        def _():
            _stage(c + 1, 1 - par)

        for r in range(IC):
            if r > 0:
                _drain_scatter(par, r - 1)
            pltpu.async_copy(ones_v, acc.at[dchunk.at[par, r]], ssem0,
                             add=True)
        _drain_scatter(par, IC - 1)

    def _body(jo, carry):
        c0 = jo * 2
        _run_chunk(c0, 0, c0 + 1 < NCHUNK)
        _run_chunk(c0 + 1, 1, c0 + 2 < NCHUNK)
        return carry

    lax.fori_loop(0, NCHUNK // 2, _body, 0, unroll=False)
    plsc.subcore_barrier()
    pltpu.sync_copy(acc.at[pl.ds(sid * ZROWS, ZROWS)],
                    out.at[cid, pl.ds(sid * ZROWS, ZROWS)])


_SC_SCRATCH = [
    pltpu.VMEM((IC, CHUNK), jnp.int32),
    pltpu.VMEM((2, IC, CHUNK), jnp.int32),
    pltpu.VMEM((NSLOT, CHUNK, HH), jnp.float32),
    pltpu.VMEM_SHARED((ACC_ROWS, HH), jnp.float32),
    pltpu.SemaphoreType.DMA((NSLOT,)),
    pltpu.SemaphoreType.DMA((NSLOT,)),
    pltpu.SemaphoreType.DMA((2,)),
    pltpu.SemaphoreType.DMA,
]


@functools.partial(
    pl.kernel,
    out_type=[jax.ShapeDtypeStruct((NC, ACC_ROWS, HH), jnp.float32)] * 2,
    mesh=_MESH,
    compiler_params=_SC_PARAMS,
    scratch_types=_SC_SCRATCH,
)
def _layer1(pg_lo, pg_hi, pd_lo, pd_hi, sdg, ddg, sgd, dgd, ones, zeros,
            s1g, s1d,
            schunk, dchunk, bufs, acc, gsem, ssem, dsem, csem):
    cid = lax.axis_index("c")
    sid = lax.axis_index("s")
    del ones
    _seg_phase(cid, sid, pg_lo, pg_hi, sdg, ddg, zeros, s1g,
               schunk, dchunk, bufs, acc, gsem, ssem, dsem, csem)
    _seg_phase(cid, sid, pd_lo, pd_hi, sgd, dgd, zeros, s1d,
               schunk, dchunk, bufs, acc, gsem, ssem, dsem, csem)


@functools.partial(
    pl.kernel,
    out_type=[jax.ShapeDtypeStruct((NC, ACC_ROWS, HH), jnp.float32)] * 2,
    mesh=_MESH,
    compiler_params=_SC_PARAMS,
    scratch_types=_SC_SCRATCH,
)
def _layer2(pg_lo, pg_hi, pd_lo, pd_hi, sdg, ddg, sgd, dgd, ones, zeros,
            s2g, s2d,
            schunk, dchunk, bufs, acc, gsem, ssem, dsem, csem):
    cid = lax.axis_index("c")
    sid = lax.axis_index("s")
    del ones
    _seg_phase(cid, sid, pg_lo, pg_hi, sdg, ddg, zeros, s2g,
               schunk, dchunk, bufs, acc, gsem, ssem, dsem, csem)
    _seg_phase(cid, sid, pd_lo, pd_hi, sgd, dgd, zeros, s2d,
               schunk, dchunk, bufs, acc, gsem, ssem, dsem, csem)


@functools.partial(
    pl.kernel,
    out_type=jax.ShapeDtypeStruct((NC, ACC_ROWS, HH), jnp.float32),
    mesh=_MESH,
    compiler_params=_SC_PARAMS,
    scratch_types=[
        pltpu.VMEM((2, IC, CHUNK), jnp.int32),
        pltpu.VMEM((CHUNK, HH), jnp.float32),
        pltpu.VMEM_SHARED((ACC_ROWS, HH), jnp.float32),
        pltpu.SemaphoreType.DMA((2,)),
        pltpu.SemaphoreType.DMA,
    ],
)
def _counts(ddg, dgd, ones, zeros, cnt, dchunk, ones_v, acc, dsem, ssem0):
    cid = lax.axis_index("c")
    sid = lax.axis_index("s")
    pltpu.sync_copy(ones, ones_v)
    _cnt_phase(cid, sid, ddg, dgd, ones_v, zeros, cnt,
               dchunk, acc, dsem, ssem0)


def _pack_idx(v, fill):
    v = v.astype(jnp.int32)[:min(E, E_PAD)]
    pad = jnp.full((max(E_PAD - E, 0),), fill, jnp.int32)
    return jnp.concatenate([v, pad]).reshape(NS, ROWS, CHUNK)


def kernel(x_disease, x_gene,
           W_l1_dg, b_l1_dg, W_r1_dg,
           W_l1_gd, b_l1_gd, W_r1_gd,
           W_l2_dg, b_l2_dg, W_r2_dg,
           W_l2_gd, b_l2_gd, W_r2_gd,
           edge_index_dg, edge_index_gd):
    f32 = jnp.float32
    sdg = _pack_idx(edge_index_dg[0], 0)
    ddg = _pack_idx(edge_index_dg[1], N)
    sgd = _pack_idx(edge_index_gd[0], 0)
    dgd = _pack_idx(edge_index_gd[1], N)
    zeros = jnp.zeros((ZROWS, HH), f32)
    ones = jnp.ones((CHUNK, HH), f32)

    p1 = _prep(x_disease, x_gene, W_l1_dg, W_r1_dg, W_l1_gd, W_r1_gd,
               b_l1_dg.reshape(1, H), b_l1_gd.reshape(1, H))
    pg_lo, pg_hi, pd_lo, pd_hi, r1g, r1d = p1

    cnt = _counts(ddg, dgd, ones, zeros)
    cg = cnt[0, :, 0:1]
    cd = cnt[1, :, 0:1]

    (sdg1, ddg1, sgd1, dgd1), _ = (lax.optimization_barrier(
        ((sdg, ddg, sgd, dgd), cnt)))
    s1g, s1d = _layer1(pg_lo, pg_hi, pd_lo, pd_hi,
                       sdg1, ddg1, sgd1, dgd1, ones, zeros)

    p2 = _mid(s1g, s1d, cg, cd, r1g, r1d,
              W_l2_dg, W_r2_dg, W_l2_gd, W_r2_gd,
              b_l2_dg.reshape(1, H), b_l2_gd.reshape(1, H))
    qg_lo, qg_hi, qd_lo, qd_hi, r2g, r2d = p2

    s2g, s2d = _layer2(qg_lo, qg_hi, qd_lo, qd_hi,
                       sdg, ddg, sgd, dgd, ones, zeros)

    d2, g2 = _final(s2g, s2d, cg, cd, r2g, r2d)
    return (d2, g2)


# speedup vs baseline: 1.1686x; 1.0967x over previous
import functools

import jax
import jax.numpy as jnp
from jax import lax
from jax.experimental import pallas as pl
from jax.experimental.pallas import tpu as pltpu
from jax.experimental.pallas import tpu_sc as plsc

N = 50000
E = 600000
D_IN = 128
H = 64
HH = 32

NC = 2
NS = 16
CHUNK = 128
ROWS = 296
PER_TILE = ROWS * CHUNK
E_PAD = NS * PER_TILE
ACC_ROWS = 50048
ZROWS = ACC_ROWS // NS
NSLOT = 4
IC = 37
NCHUNK = ROWS // IC

_MESH = plsc.VectorSubcoreMesh(
    core_axis_name="c", subcore_axis_name="s", num_cores=NC, num_subcores=NS)
_SC_PARAMS = pltpu.CompilerParams(use_tc_tiling_on_sc=False)

BLK = 2000
GRID = N // BLK


def _row_spec(w):
    return pl.BlockSpec((BLK, w), lambda i: (i, 0))


def _full_spec(shape):
    return pl.BlockSpec(shape, lambda i: tuple(0 for _ in shape))


def _half_spec():
    return pl.BlockSpec((2, BLK, HH), lambda i: (0, i, 0))


def _cat_halves(s):
    return jnp.concatenate([s[0], s[1]], axis=1)


def _prep_body(xd, xg, wl_dg, wr_dg, wl_gd, wr_gd, b_dg, b_gd,
               pg_lo, pg_hi, pd_lo, pd_hi, rg, rd):
    pg = jnp.dot(xd[...], wl_dg[...], preferred_element_type=jnp.float32)
    pg_lo[...] = pg[:, :HH]
    pg_hi[...] = pg[:, HH:]
    pd = jnp.dot(xg[...], wl_gd[...], preferred_element_type=jnp.float32)
    pd_lo[...] = pd[:, :HH]
    pd_hi[...] = pd[:, HH:]
    rg[...] = jnp.dot(xg[...], wr_dg[...], preferred_element_type=jnp.float32) + b_dg[...]
    rd[...] = jnp.dot(xd[...], wr_gd[...], preferred_element_type=jnp.float32) + b_gd[...]


def _prep(xd, xg, wl_dg, wr_dg, wl_gd, wr_gd, b_dg, b_gd):
    f32 = jnp.float32
    return pl.pallas_call(
        _prep_body,
        grid=(GRID,),
        in_specs=[_row_spec(D_IN), _row_spec(D_IN),
                  _full_spec((D_IN, H)), _full_spec((D_IN, H)),
                  _full_spec((D_IN, H)), _full_spec((D_IN, H)),
                  _full_spec((1, H)), _full_spec((1, H))],
        out_specs=[_row_spec(HH)] * 4 + [_row_spec(H), _row_spec(H)],
        out_shape=[jax.ShapeDtypeStruct((N, HH), f32)] * 4
                  + [jax.ShapeDtypeStruct((N, H), f32)] * 2,
    )(xd, xg, wl_dg, wr_dg, wl_gd, wr_gd, b_dg, b_gd)


def _mid_body(s1, c1, r1, w_l, w_r, b_r, p_lo, p_hi, r_out):
    x1 = _cat_halves(s1) / jnp.maximum(c1[...], 1.0) + r1[...]
    p = jnp.dot(x1, w_l[...], preferred_element_type=jnp.float32)
    p_lo[...] = p[:, :HH]
    p_hi[...] = p[:, HH:]
    r_out[...] = jnp.dot(x1, w_r[...], preferred_element_type=jnp.float32) + b_r[...]


def _mid(s1, c1, r1, w_l, w_r, b_r):
    f32 = jnp.float32
    return pl.pallas_call(
        _mid_body,
        grid=(GRID,),
        in_specs=[_half_spec(), _row_spec(1), _row_spec(H),
                  _full_spec((H, H)), _full_spec((H, H)), _full_spec((1, H))],
        out_specs=[_row_spec(HH), _row_spec(HH), _row_spec(H)],
        out_shape=[jax.ShapeDtypeStruct((N, HH), f32)] * 2
                  + [jax.ShapeDtypeStruct((N, H), f32)],
    )(s1, c1, r1, w_l, w_r, b_r)


def _final_body(s2, c2, r2, o):
    o[...] = _cat_halves(s2) / jnp.maximum(c2[...], 1.0) + r2[...]


def _final(s2, c2, r2):
    return pl.pallas_call(
        _final_body,
        grid=(GRID,),
        in_specs=[_half_spec(), _row_spec(1), _row_spec(H)],
        out_specs=[_row_spec(H)],
        out_shape=[jax.ShapeDtypeStruct((N, H), jnp.float32)],
    )(s2, c2, r2)[0]


def _seg_phase(cid, sid, p_lo, p_hi, src_idx, dst_idx, zeros, out,
               schunk, dchunk, bufs, acc, gsem, ssem, dsem, csem):
    def _stage_src(c):
        pltpu.async_copy(src_idx.at[sid, pl.ds(c * IC, IC)], schunk, csem)

    def _wait_src():
        pltpu.make_async_copy(src_idx.at[sid, pl.ds(0, IC)], schunk,
                              csem).wait()

    def _stage_dst(c, par):
        pltpu.async_copy(dst_idx.at[sid, pl.ds(c * IC, IC)], dchunk.at[par],
                         dsem.at[par])

    def _wait_dst(par):
        pltpu.make_async_copy(dst_idx.at[sid, pl.ds(0, IC)], dchunk.at[par],
                              dsem.at[par]).wait()

    def _gather(r, b):
        @pl.when(cid == 0)
        def _():
            pltpu.async_copy(p_lo.at[schunk.at[r]], bufs.at[b], gsem.at[b])

        @pl.when(cid == 1)
        def _():
            pltpu.async_copy(p_hi.at[schunk.at[r]], bufs.at[b], gsem.at[b])

    def _wait_gather(b):
        pltpu.make_async_copy(p_lo.at[pl.ds(0, CHUNK)], bufs.at[b],
                              gsem.at[b]).wait()

    def _wait_scatter(pb, ppar, pr):
        pltpu.make_async_copy(bufs.at[pb], acc.at[dchunk.at[ppar, pr]],
                              ssem.at[pb]).wait()

    def _run_chunk(c, par, has_next):
        _wait_src()
        _wait_dst(par)

        @pl.when(has_next)
        def _():
            _stage_dst(c + 1, 1 - par)

        for b in range(NSLOT - 1):
            _gather(b, b)
        for r in range(IC):
            b = r % NSLOT
            _wait_gather(b)
            if r == IC - 1:
                @pl.when(has_next)
                def _():
                    _stage_src(c + 1)
            pltpu.async_copy(bufs.at[b], acc.at[dchunk.at[par, r]],
                             ssem.at[b], add=True)
            g = r + NSLOT - 1
            if g < IC:
                gb = g % NSLOT
                if g >= NSLOT:
                    _wait_scatter(gb, par, g - NSLOT)
                _gather(g, gb)
        for r in range(IC - NSLOT, IC):
            _wait_scatter(r % NSLOT, par, r)

    pltpu.sync_copy(zeros, acc.at[pl.ds(sid * ZROWS, ZROWS)])
    _stage_src(jnp.int32(0))
    _stage_dst(jnp.int32(0), 0)
    plsc.subcore_barrier()

    def _body(jo, carry):
        c0 = jo * 2
        _run_chunk(c0, 0, c0 + 1 < NCHUNK)
        _run_chunk(c0 + 1, 1, c0 + 2 < NCHUNK)
        return carry

    lax.fori_loop(0, NCHUNK // 2, _body, 0, unroll=False)
    plsc.subcore_barrier()
    pltpu.sync_copy(acc.at[pl.ds(sid * ZROWS, ZROWS)],
                    out.at[cid, pl.ds(sid * ZROWS, ZROWS)])


@functools.partial(
    pl.kernel,
    out_type=jax.ShapeDtypeStruct((NC, ACC_ROWS, HH), jnp.float32),
    mesh=_MESH,
    compiler_params=_SC_PARAMS,
    scratch_types=[
        pltpu.VMEM((IC, CHUNK), jnp.int32),
        pltpu.VMEM((2, IC, CHUNK), jnp.int32),
        pltpu.VMEM((NSLOT, CHUNK, HH), jnp.float32),
        pltpu.VMEM_SHARED((ACC_ROWS, HH), jnp.float32),
        pltpu.SemaphoreType.DMA((NSLOT,)),
        pltpu.SemaphoreType.DMA((NSLOT,)),
        pltpu.SemaphoreType.DMA((2,)),
        pltpu.SemaphoreType.DMA,
    ],
)
def _segsum(p_lo, p_hi, src_idx, dst_idx, zeros, out,
            schunk, dchunk, bufs, acc, gsem, ssem, dsem, csem):
    cid = lax.axis_index("c")
    sid = lax.axis_index("s")
    _seg_phase(cid, sid, p_lo, p_hi, src_idx, dst_idx, zeros, out,
               schunk, dchunk, bufs, acc, gsem, ssem, dsem, csem)


@functools.partial(
    pl.kernel,
    out_type=jax.ShapeDtypeStruct((NC, ACC_ROWS, HH), jnp.float32),
    mesh=_MESH,
    compiler_params=_SC_PARAMS,
    scratch_types=[
        pltpu.VMEM((2, IC, CHUNK), jnp.int32),
        pltpu.VMEM((CHUNK, HH), jnp.float32),
        pltpu.VMEM_SHARED((ACC_ROWS, HH), jnp.float32),
        pltpu.SemaphoreType.DMA((2,)),
        pltpu.SemaphoreType.DMA,
    ],
)
def _counts(ddg, dgd, ones, zeros, cnt, dchunk, ones_v, acc, dsem, ssem0):
    cid = lax.axis_index("c")
    sid = lax.axis_index("s")

    def _stage(c, par):
        @pl.when(cid == 0)
        def _():
            pltpu.async_copy(ddg.at[sid, pl.ds(c * IC, IC)], dchunk.at[par],
                             dsem.at[par])

        @pl.when(cid == 1)
        def _():
            pltpu.async_copy(dgd.at[sid, pl.ds(c * IC, IC)], dchunk.at[par],
                             dsem.at[par])

    def _wait_stage(par):
        pltpu.make_async_copy(ddg.at[sid, pl.ds(0, IC)], dchunk.at[par],
                              dsem.at[par]).wait()

    def _drain_scatter(ppar, pr):
        pltpu.make_async_copy(ones_v, acc.at[dchunk.at[ppar, pr]],
                              ssem0).wait()

    pltpu.sync_copy(ones, ones_v)
    pltpu.sync_copy(zeros, acc.at[pl.ds(sid * ZROWS, ZROWS)])
    _stage(jnp.int32(0), 0)
    plsc.subcore_barrier()

    def _run_chunk(c, par, has_next):
        _wait_stage(par)

        @pl.when(has_next)
        def _():
            _stage(c + 1, 1 - par)

        for r in range(IC):
            if r > 0:
                _drain_scatter(par, r - 1)
            pltpu.async_copy(ones_v, acc.at[dchunk.at[par, r]], ssem0,
                             add=True)
        _drain_scatter(par, IC - 1)

    def _body(jo, carry):
        c0 = jo * 2
        _run_chunk(c0, 0, c0 + 1 < NCHUNK)
        _run_chunk(c0 + 1, 1, c0 + 2 < NCHUNK)
        return carry

    lax.fori_loop(0, NCHUNK // 2, _body, 0, unroll=False)
    plsc.subcore_barrier()
    pltpu.sync_copy(acc.at[pl.ds(sid * ZROWS, ZROWS)],
                    cnt.at[cid, pl.ds(sid * ZROWS, ZROWS)])


def _pack_idx(v, fill):
    v = v.astype(jnp.int32)[:min(E, E_PAD)]
    pad = jnp.full((max(E_PAD - E, 0),), fill, jnp.int32)
    return jnp.concatenate([v, pad]).reshape(NS, ROWS, CHUNK)


def _chain(dep, idx):
    idx, _ = lax.optimization_barrier((idx, dep))
    return idx


def kernel(x_disease, x_gene,
           W_l1_dg, b_l1_dg, W_r1_dg,
           W_l1_gd, b_l1_gd, W_r1_gd,
           W_l2_dg, b_l2_dg, W_r2_dg,
           W_l2_gd, b_l2_gd, W_r2_gd,
           edge_index_dg, edge_index_gd):
    f32 = jnp.float32
    sdg = _pack_idx(edge_index_dg[0], 0)
    ddg = _pack_idx(edge_index_dg[1], N)
    sgd = _pack_idx(edge_index_gd[0], 0)
    dgd = _pack_idx(edge_index_gd[1], N)
    zeros = jnp.zeros((ZROWS, HH), f32)
    ones = jnp.ones((CHUNK, HH), f32)

    pg_lo, pg_hi, pd_lo, pd_hi, r1g, r1d = _prep(
        x_disease, x_gene, W_l1_dg, W_r1_dg, W_l1_gd, W_r1_gd,
        b_l1_dg.reshape(1, H), b_l1_gd.reshape(1, H))
    cnt = _counts(ddg, dgd, ones, zeros)
    cg = cnt[0, :, 0:1]
    cd = cnt[1, :, 0:1]

    sgd1, dgd1 = _chain(cnt, (sgd, dgd))
    s1d = _segsum(pd_lo, pd_hi, sgd1, dgd1, zeros)

    sdg1, ddg1 = _chain(s1d, (sdg, ddg))
    qg_lo, qg_hi, r2d = _mid(s1d, cd, r1d, W_l2_dg, W_r2_gd,
                             b_l2_gd.reshape(1, H))
    s1g = _segsum(pg_lo, pg_hi, sdg1, ddg1, zeros)

    sdg2, ddg2 = _chain(s1g, (sdg, ddg))
    qd_lo, qd_hi, r2g = _mid(s1g, cg, r1g, W_l2_gd, W_r2_dg,
                             b_l2_dg.reshape(1, H))
    s2g = _segsum(qg_lo, qg_hi, sdg2, ddg2, zeros)

    sgd2, dgd2 = _chain(s2g, (sgd, dgd))
    g2 = _final(s2g, cg, r2g)
    s2d = _segsum(qd_lo, qd_hi, sgd2, dgd2, zeros)

    d2 = _final(s2d, cd, r2d)
    return (d2, g2)


# speedup vs baseline: 1.1967x; 1.0241x over previous
import functools

import jax
import jax.numpy as jnp
from jax import lax
from jax.experimental import pallas as pl
from jax.experimental.pallas import tpu as pltpu
from jax.experimental.pallas import tpu_sc as plsc

N = 50000
E = 600000
D_IN = 128
H = 64
HH = 32

NC = 2
NS = 16
CHUNK = 128
ROWS = 296
PER_TILE = ROWS * CHUNK
E_PAD = NS * PER_TILE
ACC_ROWS = 50048
ZROWS = ACC_ROWS // NS
NSLOT = 5
IC = 37
NCHUNK = ROWS // IC

_MESH = plsc.VectorSubcoreMesh(
    core_axis_name="c", subcore_axis_name="s", num_cores=NC, num_subcores=NS)
_SC_PARAMS = pltpu.CompilerParams(use_tc_tiling_on_sc=False)

BLK = 2000
GRID = N // BLK


def _row_spec(w):
    return pl.BlockSpec((BLK, w), lambda i: (i, 0))


def _full_spec(shape):
    return pl.BlockSpec(shape, lambda i: tuple(0 for _ in shape))


def _half_spec():
    return pl.BlockSpec((2, BLK, HH), lambda i: (0, i, 0))


def _cat_halves(s):
    return jnp.concatenate([s[0], s[1]], axis=1)


def _prep_body(xd, xg, wl_dg, wr_dg, wl_gd, wr_gd, b_dg, b_gd,
               pg_lo, pg_hi, pd_lo, pd_hi, rg, rd):
    pg = jnp.dot(xd[...], wl_dg[...], preferred_element_type=jnp.float32)
    pg_lo[...] = pg[:, :HH]
    pg_hi[...] = pg[:, HH:]
    pd = jnp.dot(xg[...], wl_gd[...], preferred_element_type=jnp.float32)
    pd_lo[...] = pd[:, :HH]
    pd_hi[...] = pd[:, HH:]
    rg[...] = jnp.dot(xg[...], wr_dg[...], preferred_element_type=jnp.float32) + b_dg[...]
    rd[...] = jnp.dot(xd[...], wr_gd[...], preferred_element_type=jnp.float32) + b_gd[...]


def _prep(xd, xg, wl_dg, wr_dg, wl_gd, wr_gd, b_dg, b_gd):
    f32 = jnp.float32
    return pl.pallas_call(
        _prep_body,
        grid=(GRID,),
        in_specs=[_row_spec(D_IN), _row_spec(D_IN),
                  _full_spec((D_IN, H)), _full_spec((D_IN, H)),
                  _full_spec((D_IN, H)), _full_spec((D_IN, H)),
                  _full_spec((1, H)), _full_spec((1, H))],
        out_specs=[_row_spec(HH)] * 4 + [_row_spec(H), _row_spec(H)],
        out_shape=[jax.ShapeDtypeStruct((N, HH), f32)] * 4
                  + [jax.ShapeDtypeStruct((N, H), f32)] * 2,
    )(xd, xg, wl_dg, wr_dg, wl_gd, wr_gd, b_dg, b_gd)


def _mid_body(s1, c1, r1, w_l, w_r, b_r, p_lo, p_hi, r_out):
    x1 = _cat_halves(s1) / jnp.maximum(c1[...], 1.0) + r1[...]
    p = jnp.dot(x1, w_l[...], preferred_element_type=jnp.float32)
    p_lo[...] = p[:, :HH]
    p_hi[...] = p[:, HH:]
    r_out[...] = jnp.dot(x1, w_r[...], preferred_element_type=jnp.float32) + b_r[...]


def _mid(s1, c1, r1, w_l, w_r, b_r):
    f32 = jnp.float32
    return pl.pallas_call(
        _mid_body,
        grid=(GRID,),
        in_specs=[_half_spec(), _row_spec(1), _row_spec(H),
                  _full_spec((H, H)), _full_spec((H, H)), _full_spec((1, H))],
        out_specs=[_row_spec(HH), _row_spec(HH), _row_spec(H)],
        out_shape=[jax.ShapeDtypeStruct((N, HH), f32)] * 2
                  + [jax.ShapeDtypeStruct((N, H), f32)],
    )(s1, c1, r1, w_l, w_r, b_r)


def _final_body(s2, c2, r2, o):
    o[...] = _cat_halves(s2) / jnp.maximum(c2[...], 1.0) + r2[...]


def _final(s2, c2, r2):
    return pl.pallas_call(
        _final_body,
        grid=(GRID,),
        in_specs=[_half_spec(), _row_spec(1), _row_spec(H)],
        out_specs=[_row_spec(H)],
        out_shape=[jax.ShapeDtypeStruct((N, H), jnp.float32)],
    )(s2, c2, r2)[0]


def _seg_phase(cid, sid, p_lo, p_hi, src_idx, dst_idx, zeros, out,
               schunk, dchunk, bufs, acc, gsem, ssem, dsem, csem):
    def _stage_src(c):
        pltpu.async_copy(src_idx.at[sid, pl.ds(c * IC, IC)], schunk, csem)

    def _wait_src():
        pltpu.make_async_copy(src_idx.at[sid, pl.ds(0, IC)], schunk,
                              csem).wait()

    def _stage_dst(c):
        pltpu.async_copy(dst_idx.at[sid, pl.ds(c * IC, IC)], dchunk, dsem)

    def _wait_dst():
        pltpu.make_async_copy(dst_idx.at[sid, pl.ds(0, IC)], dchunk,
                              dsem).wait()

    def _gather(r, b):
        @pl.when(cid == 0)
        def _():
            pltpu.async_copy(p_lo.at[schunk.at[r]], bufs.at[b], gsem.at[b])

        @pl.when(cid == 1)
        def _():
            pltpu.async_copy(p_hi.at[schunk.at[r]], bufs.at[b], gsem.at[b])

    def _wait_gather(b):
        pltpu.make_async_copy(p_lo.at[pl.ds(0, CHUNK)], bufs.at[b],
                              gsem.at[b]).wait()

    def _wait_scatter(pb, pr):
        pltpu.make_async_copy(bufs.at[pb], acc.at[dchunk.at[pr]],
                              ssem.at[pb]).wait()

    def _run_chunk(c, has_next):
        _wait_src()
        _wait_dst()

        for b in range(NSLOT - 1):
            _gather(b, b)
        for r in range(IC):
            b = r % NSLOT
            _wait_gather(b)
            if r == IC - 1:
                @pl.when(has_next)
                def _():
                    _stage_src(c + 1)
            pltpu.async_copy(bufs.at[b], acc.at[dchunk.at[r]],
                             ssem.at[b], add=True)
            g = r + NSLOT - 1
            if g < IC:
                gb = g % NSLOT
                if g >= NSLOT:
                    _wait_scatter(gb, g - NSLOT)
                _gather(g, gb)
        for r in range(IC - NSLOT, IC):
            _wait_scatter(r % NSLOT, r)

        @pl.when(has_next)
        def _():
            _stage_dst(c + 1)

    pltpu.sync_copy(zeros, acc.at[pl.ds(sid * ZROWS, ZROWS)])
    _stage_src(jnp.int32(0))
    _stage_dst(jnp.int32(0))
    plsc.subcore_barrier()

    def _body(c, carry):
        _run_chunk(c, c + 1 < NCHUNK)
        return carry

    lax.fori_loop(0, NCHUNK, _body, 0, unroll=False)
    plsc.subcore_barrier()
    pltpu.sync_copy(acc.at[pl.ds(sid * ZROWS, ZROWS)],
                    out.at[cid, pl.ds(sid * ZROWS, ZROWS)])


@functools.partial(
    pl.kernel,
    out_type=jax.ShapeDtypeStruct((NC, ACC_ROWS, HH), jnp.float32),
    mesh=_MESH,
    compiler_params=_SC_PARAMS,
    scratch_types=[
        pltpu.VMEM((IC, CHUNK), jnp.int32),
        pltpu.VMEM((IC, CHUNK), jnp.int32),
        pltpu.VMEM((NSLOT, CHUNK, HH), jnp.float32),
        pltpu.VMEM_SHARED((ACC_ROWS, HH), jnp.float32),
        pltpu.SemaphoreType.DMA((NSLOT,)),
        pltpu.SemaphoreType.DMA((NSLOT,)),
        pltpu.SemaphoreType.DMA,
        pltpu.SemaphoreType.DMA,
    ],
)
def _segsum(p_lo, p_hi, src_idx, dst_idx, zeros, out,
            schunk, dchunk, bufs, acc, gsem, ssem, dsem, csem):
    cid = lax.axis_index("c")
    sid = lax.axis_index("s")
    _seg_phase(cid, sid, p_lo, p_hi, src_idx, dst_idx, zeros, out,
               schunk, dchunk, bufs, acc, gsem, ssem, dsem, csem)


@functools.partial(
    pl.kernel,
    out_type=jax.ShapeDtypeStruct((NC, ACC_ROWS, HH), jnp.float32),
    mesh=_MESH,
    compiler_params=_SC_PARAMS,
    scratch_types=[
        pltpu.VMEM((2, IC, CHUNK), jnp.int32),
        pltpu.VMEM((CHUNK, HH), jnp.float32),
        pltpu.VMEM_SHARED((ACC_ROWS, HH), jnp.float32),
        pltpu.SemaphoreType.DMA((2,)),
        pltpu.SemaphoreType.DMA,
    ],
)
def _counts(ddg, dgd, ones, zeros, cnt, dchunk, ones_v, acc, dsem, ssem0):
    cid = lax.axis_index("c")
    sid = lax.axis_index("s")

    def _stage(c, par):
        @pl.when(cid == 0)
        def _():
            pltpu.async_copy(ddg.at[sid, pl.ds(c * IC, IC)], dchunk.at[par],
                             dsem.at[par])

        @pl.when(cid == 1)
        def _():
            pltpu.async_copy(dgd.at[sid, pl.ds(c * IC, IC)], dchunk.at[par],
                             dsem.at[par])

    def _wait_stage(par):
        pltpu.make_async_copy(ddg.at[sid, pl.ds(0, IC)], dchunk.at[par],
                              dsem.at[par]).wait()

    def _drain_scatter(ppar, pr):
        pltpu.make_async_copy(ones_v, acc.at[dchunk.at[ppar, pr]],
                              ssem0).wait()

    pltpu.sync_copy(ones, ones_v)
    pltpu.sync_copy(zeros, acc.at[pl.ds(sid * ZROWS, ZROWS)])
    _stage(jnp.int32(0), 0)
    plsc.subcore_barrier()

    def _run_chunk(c, par, has_next):
        _wait_stage(par)

        @pl.when(has_next)
        def _():
            _stage(c + 1, 1 - par)

        for r in range(IC):
            if r > 0:
                _drain_scatter(par, r - 1)
            pltpu.async_copy(ones_v, acc.at[dchunk.at[par, r]], ssem0,
                             add=True)
        _drain_scatter(par, IC - 1)

    def _body(jo, carry):
        c0 = jo * 2
        _run_chunk(c0, 0, c0 + 1 < NCHUNK)
        _run_chunk(c0 + 1, 1, c0 + 2 < NCHUNK)
        return carry

    lax.fori_loop(0, NCHUNK // 2, _body, 0, unroll=False)
    plsc.subcore_barrier()
    pltpu.sync_copy(acc.at[pl.ds(sid * ZROWS, ZROWS)],
                    cnt.at[cid, pl.ds(sid * ZROWS, ZROWS)])


def _pack_idx(v, fill):
    v = v.astype(jnp.int32)[:min(E, E_PAD)]
    pad = jnp.full((max(E_PAD - E, 0),), fill, jnp.int32)
    return jnp.concatenate([v, pad]).reshape(NS, ROWS, CHUNK)


def _chain(dep, idx):
    idx, _ = lax.optimization_barrier((idx, dep))
    return idx


def kernel(x_disease, x_gene,
           W_l1_dg, b_l1_dg, W_r1_dg,
           W_l1_gd, b_l1_gd, W_r1_gd,
           W_l2_dg, b_l2_dg, W_r2_dg,
           W_l2_gd, b_l2_gd, W_r2_gd,
           edge_index_dg, edge_index_gd):
    f32 = jnp.float32
    sdg = _pack_idx(edge_index_dg[0], 0)
    ddg = _pack_idx(edge_index_dg[1], N)
    sgd = _pack_idx(edge_index_gd[0], 0)
    dgd = _pack_idx(edge_index_gd[1], N)
    zeros = jnp.zeros((ZROWS, HH), f32)
    ones = jnp.ones((CHUNK, HH), f32)

    pg_lo, pg_hi, pd_lo, pd_hi, r1g, r1d = _prep(
        x_disease, x_gene, W_l1_dg, W_r1_dg, W_l1_gd, W_r1_gd,
        b_l1_dg.reshape(1, H), b_l1_gd.reshape(1, H))
    cnt = _counts(ddg, dgd, ones, zeros)
    cg = cnt[0, :, 0:1]
    cd = cnt[1, :, 0:1]

    sgd1, dgd1 = _chain(cnt, (sgd, dgd))
    s1d = _segsum(pd_lo, pd_hi, sgd1, dgd1, zeros)

    sdg1, ddg1 = _chain(s1d, (sdg, ddg))
    qg_lo, qg_hi, r2d = _mid(s1d, cd, r1d, W_l2_dg, W_r2_gd,
                             b_l2_gd.reshape(1, H))
    s1g = _segsum(pg_lo, pg_hi, sdg1, ddg1, zeros)

    sdg2, ddg2 = _chain(s1g, (sdg, ddg))
    qd_lo, qd_hi, r2g = _mid(s1g, cg, r1g, W_l2_gd, W_r2_dg,
                             b_l2_dg.reshape(1, H))
    s2g = _segsum(qg_lo, qg_hi, sdg2, ddg2, zeros)

    sgd2, dgd2 = _chain(s2g, (sgd, dgd))
    g2 = _final(s2g, cg, r2g)
    s2d = _segsum(qd_lo, qd_hi, sgd2, dgd2, zeros)

    d2 = _final(s2d, cd, r2d)
    return (d2, g2)
